# baseline (speedup 1.0000x reference)
import functools

import jax
import jax.numpy as jnp
import numpy as np
from jax import lax
from jax.experimental import pallas as pl
from jax.experimental.pallas import tpu as pltpu
from jax.experimental.pallas import tpu_sc as plsc

_NUM_ANGLE = 100
_NUM_RHO = 100
_ASC = 24
_GRP = 12
_NSC = 2
_NTILE = 16
_CB = 128
_STRIP = 640
_PXCHUNK = 128
_NCHUNK = _STRIP // _PXCHUNK
_RHO_PAD = 128
_ANGLE_BLK = 4


def _bin_table(H, W, numangle, numrho):
    irho = float(int(np.sqrt(H * H + W * W) + 1)) / float(numrho - 1)
    itheta = np.pi / numangle
    angles = np.arange(numangle, dtype=np.float64) * itheta
    tab_cos = (np.cos(angles) / irho).astype(np.float32)
    tab_sin = (np.sin(angles) / irho).astype(np.float32)
    ys, xs = np.meshgrid(np.arange(H), np.arange(W), indexing="ij")
    xx = (xs - (W // 2)).reshape(-1).astype(np.float32)
    yy = (ys - (H // 2)).reshape(-1).astype(np.float32)
    proj = xx[None, :] * tab_cos[:, None] + yy[None, :] * tab_sin[:, None]
    proj = proj.astype(np.float32)
    r = np.where(proj >= 0, np.floor(proj + 0.5), np.ceil(proj - 0.5))
    r = r.astype(np.int32) + (numrho // 2)
    return np.clip(r, 0, numrho - 1)


def _sc_body(featT, idx_hbm, zeros_hbm, out_hbm, feat_v, idx_v, acc_sh,
             sem_sc, *, num_cb, bins_per_tile):
    c = lax.axis_index("c")
    t = lax.axis_index("s")
    col0 = c * (num_cb * _CB)
    for cb in range(num_cb):
        colo = col0 + cb * _CB
        pltpu.sync_copy(zeros_hbm, acc_sh.at[pl.ds(t * bins_per_tile, bins_per_tile)])
        plsc.subcore_barrier()

        for k in range(_NCHUNK):
            pltpu.sync_copy(
                featT.at[pl.ds(t * _STRIP + k * _PXCHUNK, _PXCHUNK), pl.ds(colo, _CB)],
                feat_v)
            pltpu.sync_copy(idx_hbm.at[t, k], idx_v)

            def group_step(i, carry):
                descs = []
                for g in range(_GRP):
                    d = pltpu.async_copy(
                        feat_v,
                        acc_sh.at[idx_v.at[i * _GRP + g]],
                        sem_sc, add=True)
                    descs.append(d)
                for d in descs:
                    d.wait()
                return carry

            lax.fori_loop(0, _ASC // _GRP, group_step, 0)
        plsc.subcore_barrier()
        pltpu.sync_copy(
            acc_sh.at[pl.ds(t * bins_per_tile, bins_per_tile)],
            out_hbm.at[pl.ds(t * bins_per_tile, bins_per_tile), pl.ds(colo, _CB)])
        plsc.subcore_barrier()


def _tc_body(r_ref, feat_ref, out_ref, *, pp):
    i128 = lax.broadcasted_iota(jnp.int32, (pp, _RHO_PAD), 1)
    ohs = [(r_ref[i, 0, :][:, None] == i128).astype(jnp.bfloat16)
           for i in range(_ANGLE_BLK)]
    oh = jnp.concatenate(ohs, axis=1)
    acc = jax.lax.dot_general(
        feat_ref[...], oh,
        dimension_numbers=(((1,), (0,)), ((), ())),
        preferred_element_type=jnp.float32,
    )
    out_ref[:, 0] = acc.reshape(acc.shape[0], _ANGLE_BLK, _RHO_PAD)


def kernel(feat):
    N, C, H, W = feat.shape
    NC = N * C
    P = H * W
    PP = _NTILE * _STRIP
    A, R = _NUM_ANGLE, _NUM_RHO
    ATC = A - _ASC
    BINS_SC = _ASC * R
    bins_per_tile = -(-BINS_SC // (_NTILE * 8)) * 8
    BINSP = _NTILE * bins_per_tile

    r_np = _bin_table(H, W, A, R)

    bins_np = np.zeros((_ASC, PP), dtype=np.int32)
    bins_np[:, :P] = r_np[:_ASC] + (np.arange(_ASC, dtype=np.int32) * R)[:, None]
    idx_np = np.ascontiguousarray(
        bins_np.reshape(_ASC, _NTILE, _NCHUNK, _PXCHUNK).transpose(1, 2, 0, 3))
    idx_tab = jnp.asarray(idx_np)

    featT = jnp.pad(feat.reshape(NC, P).T, ((0, PP - P), (0, 0)))
    zeros = jnp.zeros((bins_per_tile, _CB), jnp.float32)

    mesh = plsc.VectorSubcoreMesh(
        core_axis_name="c", subcore_axis_name="s",
        num_cores=_NSC, num_subcores=_NTILE)
    num_cb = NC // (_NSC * _CB)
    sc_body = functools.partial(_sc_body, num_cb=num_cb, bins_per_tile=bins_per_tile)
    out_sc = pl.kernel(
        sc_body,
        out_type=jax.ShapeDtypeStruct((BINSP, NC), jnp.float32),
        mesh=mesh,
        scratch_types=[
            pltpu.VMEM((_PXCHUNK, _CB), jnp.float32),
            pltpu.VMEM((_ASC, _PXCHUNK), jnp.int32),
            pltpu.VMEM_SHARED((BINSP, _CB), jnp.float32),
            pltpu.SemaphoreType.DMA,
        ],
    )(featT, idx_tab, zeros)

    r_pad = np.full((ATC, 1, PP), R, dtype=np.int32)
    r_pad[:, 0, :P] = r_np[_ASC:]
    r_tab = jnp.asarray(r_pad)

    feat2 = feat.reshape(NC, P).astype(jnp.bfloat16)
    feat2 = jnp.pad(feat2, ((0, 0), (0, PP - P)))

    out_tc = pl.pallas_call(
        functools.partial(_tc_body, pp=PP),
        grid=(ATC // _ANGLE_BLK,),
        in_specs=[
            pl.BlockSpec((_ANGLE_BLK, 1, PP), lambda a: (a, 0, 0)),
            pl.BlockSpec((NC, PP), lambda a: (0, 0)),
        ],
        out_specs=pl.BlockSpec((NC, 1, _ANGLE_BLK, _RHO_PAD), lambda a: (0, a, 0, 0)),
        out_shape=jax.ShapeDtypeStruct((NC, ATC // _ANGLE_BLK, _ANGLE_BLK, _RHO_PAD), jnp.float32),
    )(r_tab, feat2)

    o_sc = out_sc[:BINS_SC].T.reshape(NC, _ASC, R)
    o_tc = out_tc[:, :, :, :R].reshape(NC, ATC, R)
    out = jnp.concatenate([o_sc, o_tc], axis=1)
    return out.reshape(N, C, A, R)


# speedup vs baseline: 1.4957x; 1.4957x over previous
import functools

import jax
import jax.numpy as jnp
import numpy as np
from jax import lax
from jax.experimental import pallas as pl
from jax.experimental.pallas import tpu as pltpu
from jax.experimental.pallas import tpu_sc as plsc

_NUM_ANGLE = 100
_NUM_RHO = 100
_ASC = 24
_GRP = 8
_NSC = 2
_NTILE = 16
_CB = 128
_STRIP = 640
_PXCHUNK = 128
_NCHUNK = _STRIP // _PXCHUNK
_RHO_PAD = 128
_ANGLE_BLK = 2


def _bin_table(H, W, numangle, numrho):
    irho = float(int(np.sqrt(H * H + W * W) + 1)) / float(numrho - 1)
    itheta = np.pi / numangle
    angles = np.arange(numangle, dtype=np.float64) * itheta
    tab_cos = (np.cos(angles) / irho).astype(np.float32)
    tab_sin = (np.sin(angles) / irho).astype(np.float32)
    ys, xs = np.meshgrid(np.arange(H), np.arange(W), indexing="ij")
    xx = (xs - (W // 2)).reshape(-1).astype(np.float32)
    yy = (ys - (H // 2)).reshape(-1).astype(np.float32)
    proj = xx[None, :] * tab_cos[:, None] + yy[None, :] * tab_sin[:, None]
    proj = proj.astype(np.float32)
    r = np.where(proj >= 0, np.floor(proj + 0.5), np.ceil(proj - 0.5))
    r = r.astype(np.int32) + (numrho // 2)
    return np.clip(r, 0, numrho - 1)


def _sc_body(featT, idx_hbm, zeros_hbm, out_hbm, feat_v, idx_v, acc_sh,
             sem_sc, *, num_cb, bins_per_tile):
    c = lax.axis_index("c")
    t = lax.axis_index("s")
    col0 = c * (num_cb * _CB)
    for cb in range(num_cb):
        colo = col0 + cb * _CB
        pltpu.sync_copy(zeros_hbm, acc_sh.at[pl.ds(t * bins_per_tile, bins_per_tile)])
        plsc.subcore_barrier()

        for k in range(_NCHUNK):
            pltpu.sync_copy(
                featT.at[pl.ds(t * _STRIP + k * _PXCHUNK, _PXCHUNK), pl.ds(colo, _CB)],
                feat_v)
            pltpu.sync_copy(idx_hbm.at[t, k], idx_v)

            def group_step(i, carry):
                descs = []
                for g in range(_GRP):
                    d = pltpu.async_copy(
                        feat_v,
                        acc_sh.at[idx_v.at[i * _GRP + g]],
                        sem_sc, add=True)
                    descs.append(d)
                for d in descs:
                    d.wait()
                return carry

            lax.fori_loop(0, _ASC // _GRP, group_step, 0)
        plsc.subcore_barrier()
        pltpu.sync_copy(
            acc_sh.at[pl.ds(t * bins_per_tile, bins_per_tile)],
            out_hbm.at[pl.ds(t * bins_per_tile, bins_per_tile), pl.ds(colo, _CB)])
        plsc.subcore_barrier()


def _tc_body(r_ref, feat_ref, out_ref, *, pp):
    i128 = lax.broadcasted_iota(jnp.int32, (pp, _RHO_PAD), 1)
    ohs = [(r_ref[i, 0, :][:, None] == i128).astype(jnp.bfloat16)
           for i in range(_ANGLE_BLK)]
    oh = jnp.concatenate(ohs, axis=1)
    out_ref[0] = jax.lax.dot_general(
        feat_ref[...], oh,
        dimension_numbers=(((1,), (0,)), ((), ())),
        preferred_element_type=jnp.float32,
    )


def kernel(feat):
    N, C, H, W = feat.shape
    NC = N * C
    P = H * W
    PP = _NTILE * _STRIP
    A, R = _NUM_ANGLE, _NUM_RHO
    ATC = A - _ASC
    BINS_SC = _ASC * R
    bins_per_tile = -(-BINS_SC // (_NTILE * 8)) * 8
    BINSP = _NTILE * bins_per_tile

    r_np = _bin_table(H, W, A, R)

    bins_np = np.zeros((_ASC, PP), dtype=np.int32)
    bins_np[:, :P] = r_np[:_ASC] + (np.arange(_ASC, dtype=np.int32) * R)[:, None]
    idx_np = np.ascontiguousarray(
        bins_np.reshape(_ASC, _NTILE, _NCHUNK, _PXCHUNK).transpose(1, 2, 0, 3))
    idx_tab = jnp.asarray(idx_np)

    featT = jnp.pad(feat.reshape(NC, P).T, ((0, PP - P), (0, 0)))
    zeros = jnp.zeros((bins_per_tile, _CB), jnp.float32)

    mesh = plsc.VectorSubcoreMesh(
        core_axis_name="c", subcore_axis_name="s",
        num_cores=_NSC, num_subcores=_NTILE)
    num_cb = NC // (_NSC * _CB)
    sc_body = functools.partial(_sc_body, num_cb=num_cb, bins_per_tile=bins_per_tile)
    out_sc = pl.kernel(
        sc_body,
        out_type=jax.ShapeDtypeStruct((BINSP, NC), jnp.float32),
        mesh=mesh,
        scratch_types=[
            pltpu.VMEM((_PXCHUNK, _CB), jnp.float32),
            pltpu.VMEM((_ASC, _PXCHUNK), jnp.int32),
            pltpu.VMEM_SHARED((BINSP, _CB), jnp.float32),
            pltpu.SemaphoreType.DMA,
        ],
    )(featT, idx_tab, zeros)

    r_pad = np.full((ATC, 1, PP), R, dtype=np.int32)
    r_pad[:, 0, :P] = r_np[_ASC:]
    r_tab = jnp.asarray(r_pad)

    feat2 = feat.reshape(NC, P).astype(jnp.bfloat16)
    feat2 = jnp.pad(feat2, ((0, 0), (0, PP - P)))

    out_tc = pl.pallas_call(
        functools.partial(_tc_body, pp=PP),
        grid=(ATC // _ANGLE_BLK,),
        in_specs=[
            pl.BlockSpec((_ANGLE_BLK, 1, PP), lambda a: (a, 0, 0)),
            pl.BlockSpec((NC, PP), lambda a: (0, 0)),
        ],
        out_specs=pl.BlockSpec((1, NC, _ANGLE_BLK * _RHO_PAD), lambda a: (a, 0, 0)),
        out_shape=jax.ShapeDtypeStruct((ATC // _ANGLE_BLK, NC, _ANGLE_BLK * _RHO_PAD), jnp.float32),
    )(r_tab, feat2)

    o_sc = out_sc[:BINS_SC].T.reshape(NC, _ASC, R)
    o_tc = out_tc.reshape(ATC // _ANGLE_BLK, NC, _ANGLE_BLK, _RHO_PAD)[:, :, :, :R]
    o_tc = o_tc.transpose(1, 0, 2, 3).reshape(NC, ATC, R)
    out = jnp.concatenate([o_sc, o_tc], axis=1)
    return out.reshape(N, C, A, R)


# speedup vs baseline: 1.5906x; 1.0634x over previous
import functools

import jax
import jax.numpy as jnp
import numpy as np
from jax import lax
from jax.experimental import pallas as pl
from jax.experimental.pallas import tpu as pltpu
from jax.experimental.pallas import tpu_sc as plsc

_NUM_ANGLE = 100
_NUM_RHO = 100
_ASC = 16
_GRP = 8
_NSC = 2
_NTILE = 16
_CB = 128
_STRIP = 640
_PXCHUNK = 128
_NCHUNK = _STRIP // _PXCHUNK
_RHO_PAD = 128
_ANGLE_BLK = 2


def _bin_table(H, W, numangle, numrho):
    irho = float(int(np.sqrt(H * H + W * W) + 1)) / float(numrho - 1)
    itheta = np.pi / numangle
    angles = np.arange(numangle, dtype=np.float64) * itheta
    tab_cos = (np.cos(angles) / irho).astype(np.float32)
    tab_sin = (np.sin(angles) / irho).astype(np.float32)
    ys, xs = np.meshgrid(np.arange(H), np.arange(W), indexing="ij")
    xx = (xs - (W // 2)).reshape(-1).astype(np.float32)
    yy = (ys - (H // 2)).reshape(-1).astype(np.float32)
    proj = xx[None, :] * tab_cos[:, None] + yy[None, :] * tab_sin[:, None]
    proj = proj.astype(np.float32)
    r = np.where(proj >= 0, np.floor(proj + 0.5), np.ceil(proj - 0.5))
    r = r.astype(np.int32) + (numrho // 2)
    return np.clip(r, 0, numrho - 1)


def _sc_body(featT, idx_hbm, zeros_hbm, out_hbm, feat_v, idx_v, acc_sh,
             sem_sc, *, num_cb, bins_per_tile):
    c = lax.axis_index("c")
    t = lax.axis_index("s")
    col0 = c * (num_cb * _CB)
    for cb in range(num_cb):
        colo = col0 + cb * _CB
        pltpu.sync_copy(zeros_hbm, acc_sh.at[pl.ds(t * bins_per_tile, bins_per_tile)])
        plsc.subcore_barrier()

        for k in range(_NCHUNK):
            pltpu.sync_copy(
                featT.at[pl.ds(t * _STRIP + k * _PXCHUNK, _PXCHUNK), pl.ds(colo, _CB)],
                feat_v)
            pltpu.sync_copy(idx_hbm.at[t, k], idx_v)

            def group_step(i, carry):
                descs = []
                for g in range(_GRP):
                    d = pltpu.async_copy(
                        feat_v,
                        acc_sh.at[idx_v.at[i * _GRP + g]],
                        sem_sc, add=True)
                    descs.append(d)
                for d in descs:
                    d.wait()
                return carry

            lax.fori_loop(0, _ASC // _GRP, group_step, 0)
        plsc.subcore_barrier()
        pltpu.sync_copy(
            acc_sh.at[pl.ds(t * bins_per_tile, bins_per_tile)],
            out_hbm.at[pl.ds(t * bins_per_tile, bins_per_tile), pl.ds(colo, _CB)])
        plsc.subcore_barrier()


def _tc_body(r_ref, feat_ref, out_ref, *, pp):
    i128 = lax.broadcasted_iota(jnp.int32, (pp, _RHO_PAD), 1)
    ohs = [(r_ref[i, 0, :][:, None] == i128).astype(jnp.bfloat16)
           for i in range(_ANGLE_BLK)]
    oh = jnp.concatenate(ohs, axis=1)
    out_ref[0] = jax.lax.dot_general(
        feat_ref[...], oh,
        dimension_numbers=(((1,), (0,)), ((), ())),
        preferred_element_type=jnp.float32,
    )


def kernel(feat):
    N, C, H, W = feat.shape
    NC = N * C
    P = H * W
    PP = _NTILE * _STRIP
    A, R = _NUM_ANGLE, _NUM_RHO
    ATC = A - _ASC
    BINS_SC = _ASC * R
    bins_per_tile = -(-BINS_SC // (_NTILE * 8)) * 8
    BINSP = _NTILE * bins_per_tile

    r_np = _bin_table(H, W, A, R)

    bins_np = np.zeros((_ASC, PP), dtype=np.int32)
    bins_np[:, :P] = r_np[:_ASC] + (np.arange(_ASC, dtype=np.int32) * R)[:, None]
    idx_np = np.ascontiguousarray(
        bins_np.reshape(_ASC, _NTILE, _NCHUNK, _PXCHUNK).transpose(1, 2, 0, 3))
    idx_tab = jnp.asarray(idx_np)

    featT = jnp.pad(feat.reshape(NC, P).T, ((0, PP - P), (0, 0)))
    zeros = jnp.zeros((bins_per_tile, _CB), jnp.float32)

    mesh = plsc.VectorSubcoreMesh(
        core_axis_name="c", subcore_axis_name="s",
        num_cores=_NSC, num_subcores=_NTILE)
    num_cb = NC // (_NSC * _CB)
    sc_body = functools.partial(_sc_body, num_cb=num_cb, bins_per_tile=bins_per_tile)
    out_sc = pl.kernel(
        sc_body,
        out_type=jax.ShapeDtypeStruct((BINSP, NC), jnp.float32),
        mesh=mesh,
        scratch_types=[
            pltpu.VMEM((_PXCHUNK, _CB), jnp.float32),
            pltpu.VMEM((_ASC, _PXCHUNK), jnp.int32),
            pltpu.VMEM_SHARED((BINSP, _CB), jnp.float32),
            pltpu.SemaphoreType.DMA,
        ],
    )(featT, idx_tab, zeros)

    r_pad = np.full((ATC, 1, PP), R, dtype=np.int32)
    r_pad[:, 0, :P] = r_np[_ASC:]
    r_tab = jnp.asarray(r_pad)

    feat2 = feat.reshape(NC, P).astype(jnp.bfloat16)
    feat2 = jnp.pad(feat2, ((0, 0), (0, PP - P)))

    out_tc = pl.pallas_call(
        functools.partial(_tc_body, pp=PP),
        grid=(ATC // _ANGLE_BLK,),
        in_specs=[
            pl.BlockSpec((_ANGLE_BLK, 1, PP), lambda a: (a, 0, 0)),
            pl.BlockSpec((NC, PP), lambda a: (0, 0)),
        ],
        out_specs=pl.BlockSpec((1, NC, _ANGLE_BLK * _RHO_PAD), lambda a: (a, 0, 0)),
        out_shape=jax.ShapeDtypeStruct((ATC // _ANGLE_BLK, NC, _ANGLE_BLK * _RHO_PAD), jnp.float32),
    )(r_tab, feat2)

    o_sc = out_sc[:BINS_SC].T.reshape(NC, _ASC, R)
    o_tc = out_tc.reshape(ATC // _ANGLE_BLK, NC, _ANGLE_BLK, _RHO_PAD)[:, :, :, :R]
    o_tc = o_tc.transpose(1, 0, 2, 3).reshape(NC, ATC, R)
    out = jnp.concatenate([o_sc, o_tc], axis=1)
    return out.reshape(N, C, A, R)


# speedup vs baseline: 1.6027x; 1.0076x over previous
import functools

import jax
import jax.numpy as jnp
import numpy as np
from jax import lax
from jax.experimental import pallas as pl
from jax.experimental.pallas import tpu as pltpu
from jax.experimental.pallas import tpu_sc as plsc

_NUM_ANGLE = 100
_NUM_RHO = 100
_ASC = 16
_GRP = 8
_NSC = 2
_NTILE = 16
_CB = 128
_STRIP = 640
_PXCHUNK = 128
_NCHUNK = _STRIP // _PXCHUNK
_RHO_PAD = 128
_ANGLE_BLK = 2


def _bin_table(H, W, numangle, numrho):
    irho = float(int(np.sqrt(H * H + W * W) + 1)) / float(numrho - 1)
    itheta = np.pi / numangle
    angles = np.arange(numangle, dtype=np.float64) * itheta
    tab_cos = (np.cos(angles) / irho).astype(np.float32)
    tab_sin = (np.sin(angles) / irho).astype(np.float32)
    ys, xs = np.meshgrid(np.arange(H), np.arange(W), indexing="ij")
    xx = (xs - (W // 2)).reshape(-1).astype(np.float32)
    yy = (ys - (H // 2)).reshape(-1).astype(np.float32)
    proj = xx[None, :] * tab_cos[:, None] + yy[None, :] * tab_sin[:, None]
    proj = proj.astype(np.float32)
    r = np.where(proj >= 0, np.floor(proj + 0.5), np.ceil(proj - 0.5))
    r = r.astype(np.int32) + (numrho // 2)
    return np.clip(r, 0, numrho - 1)


def _sc_body(featT, idx_hbm, zeros_hbm, out_hbm, feat_v, idx_v, acc_sh,
             sem_sc, *, num_cb, bins_per_tile):
    c = lax.axis_index("c")
    t = lax.axis_index("s")
    col0 = c * (num_cb * _CB)
    for cb in range(num_cb):
        colo = col0 + cb * _CB
        pltpu.sync_copy(zeros_hbm, acc_sh.at[pl.ds(t * bins_per_tile, bins_per_tile)])
        plsc.subcore_barrier()

        for k in range(_NCHUNK):
            pltpu.sync_copy(
                featT.at[pl.ds(t * _STRIP + k * _PXCHUNK, _PXCHUNK), pl.ds(colo, _CB)],
                feat_v)
            pltpu.sync_copy(idx_hbm.at[t, k], idx_v)

            def group_step(i, carry):
                descs = []
                for g in range(_GRP):
                    d = pltpu.async_copy(
                        feat_v,
                        acc_sh.at[idx_v.at[i * _GRP + g]],
                        sem_sc, add=True)
                    descs.append(d)
                for d in descs:
                    d.wait()
                return carry

            lax.fori_loop(0, _ASC // _GRP, group_step, 0)
        plsc.subcore_barrier()
        pltpu.sync_copy(
            acc_sh.at[pl.ds(t * bins_per_tile, bins_per_tile)],
            out_hbm.at[pl.ds(t * bins_per_tile, bins_per_tile), pl.ds(colo, _CB)])
        plsc.subcore_barrier()


def _tc_body(r_ref, feat_ref, out_ref, *, pp):
    i128 = lax.broadcasted_iota(jnp.int32, (pp, _RHO_PAD), 1)
    ohs = [(r_ref[i, 0, :][:, None] == i128).astype(jnp.bfloat16)
           for i in range(_ANGLE_BLK)]
    oh = jnp.concatenate(ohs, axis=1)
    out_ref[...] = jax.lax.dot_general(
        feat_ref[...], oh,
        dimension_numbers=(((1,), (0,)), ((), ())),
        preferred_element_type=jnp.float32,
    )


def kernel(feat):
    N, C, H, W = feat.shape
    NC = N * C
    P = H * W
    PP = _NTILE * _STRIP
    A, R = _NUM_ANGLE, _NUM_RHO
    ATC = A - _ASC
    BINS_SC = _ASC * R
    bins_per_tile = -(-BINS_SC // (_NTILE * 8)) * 8
    BINSP = _NTILE * bins_per_tile

    r_np = _bin_table(H, W, A, R)

    bins_np = np.zeros((_ASC, PP), dtype=np.int32)
    bins_np[:, :P] = r_np[:_ASC] + (np.arange(_ASC, dtype=np.int32) * R)[:, None]
    idx_np = np.ascontiguousarray(
        bins_np.reshape(_ASC, _NTILE, _NCHUNK, _PXCHUNK).transpose(1, 2, 0, 3))
    idx_tab = jnp.asarray(idx_np)

    featT = jnp.pad(feat.reshape(NC, P).T, ((0, PP - P), (0, 0)))
    zeros = jnp.zeros((bins_per_tile, _CB), jnp.float32)

    mesh = plsc.VectorSubcoreMesh(
        core_axis_name="c", subcore_axis_name="s",
        num_cores=_NSC, num_subcores=_NTILE)
    num_cb = NC // (_NSC * _CB)
    sc_body = functools.partial(_sc_body, num_cb=num_cb, bins_per_tile=bins_per_tile)
    out_sc = pl.kernel(
        sc_body,
        out_type=jax.ShapeDtypeStruct((BINSP, NC), jnp.float32),
        mesh=mesh,
        scratch_types=[
            pltpu.VMEM((_PXCHUNK, _CB), jnp.float32),
            pltpu.VMEM((_ASC, _PXCHUNK), jnp.int32),
            pltpu.VMEM_SHARED((BINSP, _CB), jnp.float32),
            pltpu.SemaphoreType.DMA,
        ],
    )(featT, idx_tab, zeros)

    r_pad = np.empty((ATC, 1, P), dtype=np.int32)
    r_pad[:, 0, :] = r_np[_ASC:]
    r_tab = jnp.asarray(r_pad)

    feat2 = feat.reshape(NC, P).astype(jnp.bfloat16)

    nsteps = ATC // _ANGLE_BLK
    out_tc = pl.pallas_call(
        functools.partial(_tc_body, pp=P),
        grid=(nsteps,),
        in_specs=[
            pl.BlockSpec((_ANGLE_BLK, 1, P), lambda a: (a, 0, 0)),
            pl.BlockSpec((NC, P), lambda a: (0, 0)),
        ],
        out_specs=pl.BlockSpec((NC, _ANGLE_BLK * _RHO_PAD), lambda a: (0, a)),
        out_shape=jax.ShapeDtypeStruct((NC, nsteps * _ANGLE_BLK * _RHO_PAD), jnp.float32),
    )(r_tab, feat2)

    o_sc = out_sc[:BINS_SC].T.reshape(NC, _ASC, R)
    o_tc = out_tc.reshape(NC, ATC, _RHO_PAD)[:, :, :R]
    out = jnp.concatenate([o_sc, o_tc], axis=1)
    return out.reshape(N, C, A, R)


# speedup vs baseline: 1.6929x; 1.0563x over previous
import functools

import jax
import jax.numpy as jnp
import numpy as np
from jax import lax
from jax.experimental import pallas as pl
from jax.experimental.pallas import tpu as pltpu
from jax.experimental.pallas import tpu_sc as plsc

_NUM_ANGLE = 100
_NUM_RHO = 100
_ASC = 16
_GRP = 8
_NSC = 2
_NTILE = 16
_CB = 128
_STRIP = 640
_PXCHUNK = 128
_NCHUNK = _STRIP // _PXCHUNK
_RHO_PAD = 128
_ANGLE_BLK = 2


def _bin_table(H, W, numangle, numrho):
    irho = float(int(np.sqrt(H * H + W * W) + 1)) / float(numrho - 1)
    itheta = np.pi / numangle
    angles = np.arange(numangle, dtype=np.float64) * itheta
    tab_cos = (np.cos(angles) / irho).astype(np.float32)
    tab_sin = (np.sin(angles) / irho).astype(np.float32)
    ys, xs = np.meshgrid(np.arange(H), np.arange(W), indexing="ij")
    xx = (xs - (W // 2)).reshape(-1).astype(np.float32)
    yy = (ys - (H // 2)).reshape(-1).astype(np.float32)
    proj = xx[None, :] * tab_cos[:, None] + yy[None, :] * tab_sin[:, None]
    proj = proj.astype(np.float32)
    r = np.where(proj >= 0, np.floor(proj + 0.5), np.ceil(proj - 0.5))
    r = r.astype(np.int32) + (numrho // 2)
    return np.clip(r, 0, numrho - 1)


def _sc_body(featT, idx_hbm, zeros_hbm, out_hbm, feat_v, idx_v, acc_sh,
             sem_sc, *, num_cb, bins_per_tile):
    c = lax.axis_index("c")
    t = lax.axis_index("s")
    col0 = c * (num_cb * _CB)
    npx = featT.shape[0]
    nchunk_total = (npx + _PXCHUNK - 1) // _PXCHUNK
    for cb in range(num_cb):
        colo = col0 + cb * _CB
        pltpu.sync_copy(zeros_hbm, acc_sh.at[pl.ds(t * bins_per_tile, bins_per_tile)])
        plsc.subcore_barrier()

        for k in range((nchunk_total + _NTILE - 1) // _NTILE):
            c = t + k * _NTILE

            @pl.when(c < nchunk_total)
            def _chunk():
                start = jnp.minimum(c * _PXCHUNK, npx - _PXCHUNK)
                pltpu.sync_copy(
                    featT.at[pl.ds(start, _PXCHUNK), pl.ds(colo, _CB)],
                    feat_v)
                pltpu.sync_copy(idx_hbm.at[c], idx_v)

                def group_step(i, carry):
                    descs = []
                    for g in range(_GRP):
                        d = pltpu.async_copy(
                            feat_v,
                            acc_sh.at[idx_v.at[i * _GRP + g]],
                            sem_sc, add=True)
                        descs.append(d)
                    for d in descs:
                        d.wait()
                    return carry

                lax.fori_loop(0, _ASC // _GRP, group_step, 0)
        plsc.subcore_barrier()
        pltpu.sync_copy(
            acc_sh.at[pl.ds(t * bins_per_tile, bins_per_tile)],
            out_hbm.at[pl.ds(t * bins_per_tile, bins_per_tile), pl.ds(colo, _CB)])
        plsc.subcore_barrier()


def _tc_body(r_ref, feat_ref, out_ref, *, pp):
    i128 = lax.broadcasted_iota(jnp.int32, (pp, _RHO_PAD), 1)
    ohs = [(r_ref[i, 0, :][:, None] == i128).astype(jnp.bfloat16)
           for i in range(_ANGLE_BLK)]
    oh = jnp.concatenate(ohs, axis=1)
    out_ref[...] = jax.lax.dot_general(
        feat_ref[...], oh,
        dimension_numbers=(((1,), (0,)), ((), ())),
        preferred_element_type=jnp.float32,
    )


def kernel(feat):
    N, C, H, W = feat.shape
    NC = N * C
    P = H * W
    PP = _NTILE * _STRIP
    A, R = _NUM_ANGLE, _NUM_RHO
    ATC = A - _ASC
    BINS_SC = _ASC * R
    bins_per_tile = -(-BINS_SC // (_NTILE * 8)) * 8
    BINSP = _NTILE * bins_per_tile

    r_np = _bin_table(H, W, A, R)

    bins_ap = r_np[:_ASC] + (np.arange(_ASC, dtype=np.int32) * R)[:, None]
    nchunk_total = (P + _PXCHUNK - 1) // _PXCHUNK
    idx_np = np.empty((nchunk_total, _ASC, _PXCHUNK), dtype=np.int32)
    covered = 0
    for cidx in range(nchunk_total):
        start = min(cidx * _PXCHUNK, P - _PXCHUNK)
        blk = bins_ap[:, start:start + _PXCHUNK].copy()
        dup = covered - start
        if dup > 0:
            blk[:, :dup] = BINSP
        idx_np[cidx] = blk
        covered = start + _PXCHUNK
    idx_tab = jnp.asarray(idx_np)

    featT = feat.reshape(NC, P).T
    zeros = jnp.zeros((bins_per_tile, _CB), jnp.float32)

    mesh = plsc.VectorSubcoreMesh(
        core_axis_name="c", subcore_axis_name="s",
        num_cores=_NSC, num_subcores=_NTILE)
    num_cb = NC // (_NSC * _CB)
    sc_body = functools.partial(_sc_body, num_cb=num_cb, bins_per_tile=bins_per_tile)
    out_sc = pl.kernel(
        sc_body,
        out_type=jax.ShapeDtypeStruct((BINSP, NC), jnp.float32),
        mesh=mesh,
        scratch_types=[
            pltpu.VMEM((_PXCHUNK, _CB), jnp.float32),
            pltpu.VMEM((_ASC, _PXCHUNK), jnp.int32),
            pltpu.VMEM_SHARED((BINSP + 8, _CB), jnp.float32),
            pltpu.SemaphoreType.DMA,
        ],
    )(featT, idx_tab, zeros)

    r_pad = np.empty((ATC, 1, P), dtype=np.int32)
    r_pad[:, 0, :] = r_np[_ASC:]
    r_tab = jnp.asarray(r_pad)

    feat2 = feat.reshape(NC, P).astype(jnp.bfloat16)

    nsteps = ATC // _ANGLE_BLK
    out_tc = pl.pallas_call(
        functools.partial(_tc_body, pp=P),
        grid=(nsteps,),
        in_specs=[
            pl.BlockSpec((_ANGLE_BLK, 1, P), lambda a: (a, 0, 0)),
            pl.BlockSpec((NC, P), lambda a: (0, 0)),
        ],
        out_specs=pl.BlockSpec((NC, _ANGLE_BLK * _RHO_PAD), lambda a: (0, a)),
        out_shape=jax.ShapeDtypeStruct((NC, nsteps * _ANGLE_BLK * _RHO_PAD), jnp.float32),
    )(r_tab, feat2)

    o_sc = out_sc[:BINS_SC].T.reshape(NC, _ASC, R)
    o_tc = out_tc.reshape(NC, ATC, _RHO_PAD)[:, :, :R]
    out = jnp.concatenate([o_sc, o_tc], axis=1)
    return out.reshape(N, C, A, R)


# speedup vs baseline: 1.8598x; 1.0986x over previous
import functools

import jax
import jax.numpy as jnp
import numpy as np
from jax import lax
from jax.experimental import pallas as pl
from jax.experimental.pallas import tpu as pltpu
from jax.experimental.pallas import tpu_sc as plsc

_NUM_ANGLE = 100
_NUM_RHO = 100
_ASC = 16
_GRP = 8
_NSC = 2
_NTILE = 16
_CB = 128
_STRIP = 640
_PXCHUNK = 128
_NCHUNK = _STRIP // _PXCHUNK
_RHO_PAD = 128
_ANGLE_BLK = 2


def _bin_table(H, W, numangle, numrho):
    irho = float(int(np.sqrt(H * H + W * W) + 1)) / float(numrho - 1)
    itheta = np.pi / numangle
    angles = np.arange(numangle, dtype=np.float64) * itheta
    tab_cos = (np.cos(angles) / irho).astype(np.float32)
    tab_sin = (np.sin(angles) / irho).astype(np.float32)
    ys, xs = np.meshgrid(np.arange(H), np.arange(W), indexing="ij")
    xx = (xs - (W // 2)).reshape(-1).astype(np.float32)
    yy = (ys - (H // 2)).reshape(-1).astype(np.float32)
    proj = xx[None, :] * tab_cos[:, None] + yy[None, :] * tab_sin[:, None]
    proj = proj.astype(np.float32)
    r = np.where(proj >= 0, np.floor(proj + 0.5), np.ceil(proj - 0.5))
    r = r.astype(np.int32) + (numrho // 2)
    return np.clip(r, 0, numrho - 1)


def _sc_body(featT, idx_hbm, zeros_hbm, out_hbm, feat_v, idx_v, acc_sh,
             sem_sc, *, num_cb, bins_per_tile):
    c = lax.axis_index("c")
    t = lax.axis_index("s")
    col0 = c * (num_cb * _CB)
    npx = featT.shape[0]
    nchunk_total = (npx + _PXCHUNK - 1) // _PXCHUNK
    for cb in range(num_cb):
        colo = col0 + cb * _CB
        pltpu.sync_copy(zeros_hbm, acc_sh.at[pl.ds(t * bins_per_tile, bins_per_tile)])
        plsc.subcore_barrier()

        for k in range((nchunk_total + _NTILE - 1) // _NTILE):
            c = t + k * _NTILE

            @pl.when(c < nchunk_total)
            def _chunk():
                start = jnp.minimum(c * _PXCHUNK, npx - _PXCHUNK)
                pltpu.sync_copy(
                    featT.at[pl.ds(start, _PXCHUNK), pl.ds(colo, _CB)],
                    feat_v)
                pltpu.sync_copy(idx_hbm.at[c], idx_v)

                def group_step(i, carry):
                    descs = []
                    for g in range(_GRP):
                        d = pltpu.async_copy(
                            feat_v,
                            acc_sh.at[idx_v.at[i * _GRP + g]],
                            sem_sc, add=True)
                        descs.append(d)
                    for d in descs:
                        d.wait()
                    return carry

                lax.fori_loop(0, _ASC // _GRP, group_step, 0)
        plsc.subcore_barrier()
        pltpu.sync_copy(
            acc_sh.at[pl.ds(t * bins_per_tile, bins_per_tile)],
            out_hbm.at[pl.ds(t * bins_per_tile, bins_per_tile), pl.ds(colo, _CB)])
        plsc.subcore_barrier()


def _asm_body(sc_ref, tc_ref, out_ref, *, atc):
    out_ref[:, :_ASC, :] = sc_ref[...]
    for j in range(atc):
        out_ref[:, _ASC + j, :] = tc_ref[:, j * _RHO_PAD:j * _RHO_PAD + _NUM_RHO]


def _tc_body(r_ref, feat_ref, out_ref, *, pp):
    i128 = lax.broadcasted_iota(jnp.int32, (pp, _RHO_PAD), 1)
    ohs = [(r_ref[i, 0, :][:, None] == i128).astype(jnp.bfloat16)
           for i in range(_ANGLE_BLK)]
    oh = jnp.concatenate(ohs, axis=1)
    out_ref[...] = jax.lax.dot_general(
        feat_ref[...], oh,
        dimension_numbers=(((1,), (0,)), ((), ())),
        preferred_element_type=jnp.float32,
    )


def kernel(feat):
    N, C, H, W = feat.shape
    NC = N * C
    P = H * W
    PP = _NTILE * _STRIP
    A, R = _NUM_ANGLE, _NUM_RHO
    ATC = A - _ASC
    BINS_SC = _ASC * R
    bins_per_tile = -(-BINS_SC // (_NTILE * 8)) * 8
    BINSP = _NTILE * bins_per_tile

    r_np = _bin_table(H, W, A, R)

    bins_ap = r_np[:_ASC] + (np.arange(_ASC, dtype=np.int32) * R)[:, None]
    nchunk_total = (P + _PXCHUNK - 1) // _PXCHUNK
    idx_np = np.empty((nchunk_total, _ASC, _PXCHUNK), dtype=np.int32)
    covered = 0
    for cidx in range(nchunk_total):
        start = min(cidx * _PXCHUNK, P - _PXCHUNK)
        blk = bins_ap[:, start:start + _PXCHUNK].copy()
        dup = covered - start
        if dup > 0:
            blk[:, :dup] = BINSP
        idx_np[cidx] = blk
        covered = start + _PXCHUNK
    idx_tab = jnp.asarray(idx_np)

    featT = feat.reshape(NC, P).T
    zeros = jnp.zeros((bins_per_tile, _CB), jnp.float32)

    mesh = plsc.VectorSubcoreMesh(
        core_axis_name="c", subcore_axis_name="s",
        num_cores=_NSC, num_subcores=_NTILE)
    num_cb = NC // (_NSC * _CB)
    sc_body = functools.partial(_sc_body, num_cb=num_cb, bins_per_tile=bins_per_tile)
    out_sc = pl.kernel(
        sc_body,
        out_type=jax.ShapeDtypeStruct((BINSP, NC), jnp.float32),
        mesh=mesh,
        scratch_types=[
            pltpu.VMEM((_PXCHUNK, _CB), jnp.float32),
            pltpu.VMEM((_ASC, _PXCHUNK), jnp.int32),
            pltpu.VMEM_SHARED((BINSP + 8, _CB), jnp.float32),
            pltpu.SemaphoreType.DMA,
        ],
    )(featT, idx_tab, zeros)

    r_pad = np.empty((ATC, 1, P), dtype=np.int32)
    r_pad[:, 0, :] = r_np[_ASC:]
    r_tab = jnp.asarray(r_pad)

    feat2 = feat.reshape(NC, P).astype(jnp.bfloat16)

    nsteps = ATC // _ANGLE_BLK
    out_tc = pl.pallas_call(
        functools.partial(_tc_body, pp=P),
        grid=(nsteps,),
        in_specs=[
            pl.BlockSpec((_ANGLE_BLK, 1, P), lambda a: (a, 0, 0)),
            pl.BlockSpec((NC, P), lambda a: (0, 0)),
        ],
        out_specs=pl.BlockSpec((NC, _ANGLE_BLK * _RHO_PAD), lambda a: (0, a)),
        out_shape=jax.ShapeDtypeStruct((NC, nsteps * _ANGLE_BLK * _RHO_PAD), jnp.float32),
    )(r_tab, feat2)

    o_sc = out_sc[:BINS_SC].T.reshape(NC, _ASC, R)
    ncb = NC // 4
    out = pl.pallas_call(
        functools.partial(_asm_body, atc=ATC),
        grid=(4,),
        in_specs=[
            pl.BlockSpec((ncb, _ASC, R), lambda i: (i, 0, 0)),
            pl.BlockSpec((ncb, nsteps * _ANGLE_BLK * _RHO_PAD), lambda i: (i, 0)),
        ],
        out_specs=pl.BlockSpec((ncb, A, R), lambda i: (i, 0, 0)),
        out_shape=jax.ShapeDtypeStruct((NC, A, R), jnp.float32),
    )(o_sc, out_tc)
    return out.reshape(N, C, A, R)
